# Initial kernel scaffold; baseline (speedup 1.0000x reference)
import jax
import jax.numpy as jnp
from jax.experimental import pallas as pl


def kernel(x, edge_index, batch, W1, as1, ad1, b1, W2, as2, ad2, b2, W3, as3, ad3, b3, W4, as4, ad4, b4, lw1, lb1, lw2, lb2, lw3, lb3, lw4, lb4):
    raise NotImplementedError("write your pallas kernel here")


# speedup vs baseline: 35.5540x; 35.5540x over previous
import functools

import jax
import jax.numpy as jnp
from jax import lax
from jax.experimental import pallas as pl
from jax.experimental.pallas import tpu as pltpu
from jax.experimental.pallas import tpu_sc as plsc

N = 10000
E = 160000
NPAD = 10240
RB = 512
CH = 128
NCH = 84
EPT = NCH * CH
EP = 16 * EPT

LAYERS = [(336, 125, 128), (250, 75, 128), (150, 50, 128), (100, 30, 128)]


def _pack_h2(h, oc, P):
    rb = h.shape[0]
    ones = jnp.ones((rb, 1), jnp.float32)
    pad = jnp.zeros((rb, P - oc - 1), jnp.float32)
    h0 = jnp.concatenate([h[:, :oc], ones, pad], axis=1)
    h1 = jnp.concatenate([h[:, oc:], ones, pad], axis=1)
    return h0, h1


def _mm_first_body(x_ref, w_ref, amat_ref, h2_ref, a_ref, *, oc, P):
    h = jnp.dot(x_ref[...], w_ref[...], preferred_element_type=jnp.float32)
    a = jnp.dot(h, amat_ref[...], preferred_element_type=jnp.float32)
    a_ref[...] = a.T
    h0, h1 = _pack_h2(h, oc, P)
    h2_ref[0] = h0
    h2_ref[1] = h1


def _unpack_prev(o, ocp, bias):
    rb = o.shape[1]
    n0 = o[0, :, :ocp]
    d0 = jnp.broadcast_to(o[0, :, ocp:ocp + 1], (rb, ocp))
    n1 = o[1, :, :ocp]
    d1 = jnp.broadcast_to(o[1, :, ocp:ocp + 1], (rb, ocp))
    x = jnp.concatenate([n0 / (d0 + 1e-16), n1 / (d1 + 1e-16)], axis=1)
    return jnp.maximum(x + bias, 0.0)


def _mm_next_body(o_ref, bias_ref, w_ref, amat_ref, h2_ref, a_ref, *, ocp, oc, P):
    x = _unpack_prev(o_ref[...], ocp, bias_ref[...])
    h = jnp.dot(x, w_ref[...], preferred_element_type=jnp.float32)
    a = jnp.dot(h, amat_ref[...], preferred_element_type=jnp.float32)
    a_ref[...] = a.T
    h0, h1 = _pack_h2(h, oc, P)
    h2_ref[0] = h0
    h2_ref[1] = h1


def _node_matmul_first(x, w, amat, oc, P):
    ic = x.shape[1]
    body = functools.partial(_mm_first_body, oc=oc, P=P)
    return pl.pallas_call(
        body,
        grid=(NPAD // RB,),
        in_specs=[
            pl.BlockSpec((RB, ic), lambda i: (i, 0)),
            pl.BlockSpec((ic, 2 * oc), lambda i: (0, 0)),
            pl.BlockSpec((2 * oc, 4), lambda i: (0, 0)),
        ],
        out_specs=[
            pl.BlockSpec((2, RB, P), lambda i: (0, i, 0)),
            pl.BlockSpec((4, RB), lambda i: (0, i)),
        ],
        out_shape=[
            jax.ShapeDtypeStruct((2, NPAD, P), jnp.float32),
            jax.ShapeDtypeStruct((4, NPAD), jnp.float32),
        ],
    )(x, w, amat)


def _node_matmul_next(prev_out, bias, w, amat, ocp, Pp, oc, P):
    body = functools.partial(_mm_next_body, ocp=ocp, oc=oc, P=P)
    return pl.pallas_call(
        body,
        grid=(NPAD // RB,),
        in_specs=[
            pl.BlockSpec((2, RB, Pp), lambda i: (0, i, 0)),
            pl.BlockSpec((1, 2 * ocp), lambda i: (0, 0)),
            pl.BlockSpec((2 * ocp, 2 * oc), lambda i: (0, 0)),
            pl.BlockSpec((2 * oc, 4), lambda i: (0, 0)),
        ],
        out_specs=[
            pl.BlockSpec((2, RB, P), lambda i: (0, i, 0)),
            pl.BlockSpec((4, RB), lambda i: (0, i)),
        ],
        out_shape=[
            jax.ShapeDtypeStruct((2, NPAD, P), jnp.float32),
            jax.ShapeDtypeStruct((4, NPAD), jnp.float32),
        ],
    )(prev_out, bias, w, amat)


def _final_node_body(o_ref, bias_ref, x_ref, *, ocp):
    x_ref[...] = _unpack_prev(o_ref[...], ocp, bias_ref[...])


def _final_node(prev_out, bias, ocp, Pp):
    body = functools.partial(_final_node_body, ocp=ocp)
    return pl.pallas_call(
        body,
        grid=(NPAD // RB,),
        in_specs=[
            pl.BlockSpec((2, RB, Pp), lambda i: (0, i, 0)),
            pl.BlockSpec((1, 2 * ocp), lambda i: (0, 0)),
        ],
        out_specs=pl.BlockSpec((RB, 2 * ocp), lambda i: (i, 0)),
        out_shape=jax.ShapeDtypeStruct((NPAD, 2 * ocp), jnp.float32),
    )(prev_out, bias)


def _mlp_body(x_ref, w1_ref, b1_ref, w2_ref, b2_ref, w3_ref, b3_ref,
              w4_ref, b4_ref, o_ref):
    h = x_ref[...]
    h = jnp.maximum(jnp.dot(h, w1_ref[...], preferred_element_type=jnp.float32)
                    + b1_ref[...], 0.0)
    h = jnp.maximum(jnp.dot(h, w2_ref[...], preferred_element_type=jnp.float32)
                    + b2_ref[...], 0.0)
    h = jnp.maximum(jnp.dot(h, w3_ref[...], preferred_element_type=jnp.float32)
                    + b3_ref[...], 0.0)
    o_ref[...] = (jnp.dot(h, w4_ref[...], preferred_element_type=jnp.float32)
                  + b4_ref[...])


def _mlp(x, lw1, lb1, lw2, lb2, lw3, lb3, lw4, lb4):
    m = x.shape[0]
    return pl.pallas_call(
        _mlp_body,
        in_specs=[pl.BlockSpec(x.shape, lambda: (0, 0))] + [
            spec for w, b in ((lw1, lb1), (lw2, lb2), (lw3, lb3), (lw4, lb4))
            for spec in (pl.BlockSpec(w.shape, lambda: (0, 0)),
                         pl.BlockSpec((1, b.shape[0]), lambda: (0, 0)))
        ],
        out_specs=pl.BlockSpec((m, lw4.shape[1]), lambda: (0, 0)),
        out_shape=jax.ShapeDtypeStruct((m, lw4.shape[1]), jnp.float32),
    )(x, lw1, lb1.reshape(1, -1), lw2, lb2.reshape(1, -1),
      lw3, lb3.reshape(1, -1), lw4, lb4.reshape(1, -1))


def _amat(att_src, att_dst):
    oc = att_src.shape[1]
    m = jnp.zeros((2 * oc, 4), jnp.float32)
    m = m.at[:oc, 0].set(att_src[0]).at[oc:, 1].set(att_src[1])
    m = m.at[:oc, 2].set(att_dst[0]).at[oc:, 3].set(att_dst[1])
    return m


def _sc_edge(h2flat, at4, srcp, dstp, P):
    mesh = plsc.VectorSubcoreMesh(core_axis_name="c", subcore_axis_name="s",
                                  num_cores=2, num_subcores=16)
    rstripe = NPAD // 16

    @functools.partial(
        pl.kernel,
        out_type=jax.ShapeDtypeStruct((2 * NPAD, P), jnp.float32),
        mesh=mesh,
        compiler_params=pltpu.CompilerParams(needs_layout_passes=False),
        scratch_types=[
            pltpu.VMEM_SHARED((NPAD,), jnp.float32),
            pltpu.VMEM_SHARED((NPAD,), jnp.float32),
            pltpu.VMEM_SHARED((NPAD, P), jnp.float32),
            pltpu.VMEM((CH,), jnp.float32),
            pltpu.VMEM((CH,), jnp.float32),
            pltpu.VMEM((CH,), jnp.int32),
            pltpu.VMEM((CH,), jnp.int32),
            pltpu.VMEM((CH,), jnp.int32),
            pltpu.VMEM((CH,), jnp.int32),
            pltpu.VMEM((CH,), jnp.int32),
            pltpu.VMEM((CH,), jnp.int32),
            pltpu.VMEM((CH,), jnp.int32),
            pltpu.VMEM((CH,), jnp.int32),
            pltpu.VMEM((CH,), jnp.float32),
            pltpu.VMEM((CH,), jnp.float32),
            pltpu.VMEM((CH, P), jnp.float32),
            pltpu.VMEM((CH, P), jnp.float32),
            pltpu.SemaphoreType.DMA(()),
            pltpu.SemaphoreType.DMA(()),
        ],
    )
    def k(h2_hbm, a_hbm, src_hbm, dst_hbm, out_hbm,
          sha_s, sha_d, acc, asg, adg, srcb0, srcb1, dstb0, dstb1,
          sidx0, sidx1, didx0, didx1, exbuf0, exbuf1, rows0, rows1,
          sem0, sem1):
        srcb = (srcb0, srcb1)
        dstb = (dstb0, dstb1)
        sidx = (sidx0, sidx1)
        didx = (didx0, didx1)
        exbuf = (exbuf0, exbuf1)
        rows = (rows0, rows1)
        sem = (sem0, sem1)
        c = lax.axis_index("c")
        s = lax.axis_index("s")
        ebase = s * EPT
        coff = c * NPAD

        @pl.when(s == 0)
        def _stage():
            pltpu.sync_copy(a_hbm.at[c], sha_s)
            pltpu.sync_copy(a_hbm.at[2 + c], sha_d)

        @pl.loop(0, CH)
        def _zb(i):
            for v in range(P // 16):
                rows[0][i, pl.ds(v * 16, 16)] = jnp.zeros((16,), jnp.float32)

        for kk in range(rstripe // CH):
            pltpu.sync_copy(rows[0], acc.at[pl.ds(s * rstripe + kk * CH, CH)])
        plsc.subcore_barrier()

        def prefetch(j, b):
            pltpu.sync_copy(src_hbm.at[pl.ds(ebase + j * CH, CH)], srcb[b])
            pltpu.sync_copy(dst_hbm.at[pl.ds(ebase + j * CH, CH)], dstb[b])

            @pl.loop(0, CH // 16)
            def _idx(g):
                sidx[b][pl.ds(g * 16, 16)] = srcb[b][pl.ds(g * 16, 16)] + coff
                didx[b][pl.ds(g * 16, 16)] = dstb[b][pl.ds(g * 16, 16)]

            pltpu.sync_copy(sha_s.at[srcb[b]], asg)
            pltpu.sync_copy(sha_d.at[dstb[b]], adg)

            @pl.loop(0, CH // 16)
            def _ex(g):
                al = asg[pl.ds(g * 16, 16)] + adg[pl.ds(g * 16, 16)]
                al = jnp.where(al >= 0, al, 0.2 * al)
                exbuf[b][pl.ds(g * 16, 16)] = jnp.exp(al)

            pltpu.async_copy(h2_hbm.at[sidx[b]], rows[b], sem[b])

        def process(b):
            pltpu.make_async_copy(h2_hbm.at[pl.ds(0, CH)], rows[b],
                                  sem[b]).wait()

            @pl.loop(0, CH)
            def _scale(e):
                exb = plsc.load_gather(exbuf[b],
                                       [jnp.zeros((16,), jnp.int32) + e])
                for v in range(P // 16):
                    rows[b][e, pl.ds(v * 16, 16)] = (
                        rows[b][e, pl.ds(v * 16, 16)] * exb)

            pltpu.sync_copy(rows[b], acc.at[didx[b]], add=True)

        prefetch(0, 0)

        @pl.loop(0, NCH // 2)
        def _pair(t):
            j0 = 2 * t
            prefetch(j0 + 1, 1)
            process(0)
            prefetch(j0 + 2, 0)
            process(1)

        pltpu.make_async_copy(h2_hbm.at[pl.ds(0, CH)], rows[0], sem[0]).wait()

        plsc.subcore_barrier()
        pltpu.sync_copy(acc.at[pl.ds(s * rstripe, rstripe)],
                        out_hbm.at[pl.ds(coff + s * rstripe, rstripe)])

    return k(h2flat, at4, srcp, dstp)


def kernel(x, edge_index, batch, W1, as1, ad1, b1, W2, as2, ad2, b2,
           W3, as3, ad3, b3, W4, as4, ad4, b4,
           lw1, lb1, lw2, lb2, lw3, lb3, lw4, lb4):
    loop = jnp.arange(N, dtype=jnp.int32)
    dummy = jnp.full((EP + CH - E - N,), NPAD - 1, jnp.int32)
    srcp = jnp.concatenate([edge_index[0].astype(jnp.int32), loop, dummy])
    dstp = jnp.concatenate([edge_index[1].astype(jnp.int32), loop, dummy])

    xp = jnp.zeros((NPAD, x.shape[1]), x.dtype).at[:N].set(x)

    params = [(W1, as1, ad1, b1), (W2, as2, ad2, b2),
              (W3, as3, ad3, b3), (W4, as4, ad4, b4)]

    h2 = a = None
    prev = None
    for li, ((ic, oc, P), (W, asl, adl, bl)) in enumerate(zip(LAYERS, params)):
        am = _amat(asl, adl)
        if li == 0:
            h2, a = _node_matmul_first(xp, W, am, oc, P)
        else:
            ocp, Pp = LAYERS[li - 1][1], LAYERS[li - 1][2]
            h2, a = _node_matmul_next(prev, params[li - 1][3].reshape(1, -1),
                                      W, am, ocp, Pp, oc, P)
        out = _sc_edge(h2.reshape(2 * NPAD, P), a, srcp, dstp, P)
        prev = out.reshape(2, NPAD, P)

    oc4, P4 = LAYERS[3][1], LAYERS[3][2]
    x4 = _final_node(prev, b4.reshape(1, -1), oc4, P4)

    xr = x4[:N].reshape(1250, 480)
    xr = jnp.zeros((1280, 480), jnp.float32).at[:1250].set(xr)
    out = _mlp(xr, lw1, lb1, lw2, lb2, lw3, lb3, lw4, lb4)
    return out[:1250]


# speedup vs baseline: 36.4182x; 1.0243x over previous
import functools

import jax
import jax.numpy as jnp
from jax import lax
from jax.experimental import pallas as pl
from jax.experimental.pallas import tpu as pltpu
from jax.experimental.pallas import tpu_sc as plsc

N = 10000
E = 160000
NPAD = 10240
RB = 512
CH = 128
NCH = 84
EPT = NCH * CH
EP = 16 * EPT

LAYERS = [(336, 125, 128), (250, 75, 128), (150, 50, 128), (100, 30, 128)]


def _pack_h2(h, oc, P):
    rb = h.shape[0]
    ones = jnp.ones((rb, 1), jnp.float32)
    pad = jnp.zeros((rb, P - oc - 1), jnp.float32)
    h0 = jnp.concatenate([h[:, :oc], ones, pad], axis=1)
    h1 = jnp.concatenate([h[:, oc:], ones, pad], axis=1)
    return h0, h1


def _mm_first_body(x_ref, w_ref, amat_ref, h2_ref, a_ref, *, oc, P):
    h = jnp.dot(x_ref[...], w_ref[...], preferred_element_type=jnp.float32)
    a = jnp.dot(h, amat_ref[...], preferred_element_type=jnp.float32)
    a_ref[...] = a.T
    h0, h1 = _pack_h2(h, oc, P)
    h2_ref[0] = h0
    h2_ref[1] = h1


def _unpack_prev(o, ocp, bias):
    rb = o.shape[1]
    n0 = o[0, :, :ocp]
    d0 = jnp.broadcast_to(o[0, :, ocp:ocp + 1], (rb, ocp))
    n1 = o[1, :, :ocp]
    d1 = jnp.broadcast_to(o[1, :, ocp:ocp + 1], (rb, ocp))
    x = jnp.concatenate([n0 / (d0 + 1e-16), n1 / (d1 + 1e-16)], axis=1)
    return jnp.maximum(x + bias, 0.0)


def _mm_next_body(o_ref, bias_ref, w_ref, amat_ref, h2_ref, a_ref, *, ocp, oc, P):
    x = _unpack_prev(o_ref[...], ocp, bias_ref[...])
    h = jnp.dot(x, w_ref[...], preferred_element_type=jnp.float32)
    a = jnp.dot(h, amat_ref[...], preferred_element_type=jnp.float32)
    a_ref[...] = a.T
    h0, h1 = _pack_h2(h, oc, P)
    h2_ref[0] = h0
    h2_ref[1] = h1


def _node_matmul_first(x, w, amat, oc, P):
    ic = x.shape[1]
    body = functools.partial(_mm_first_body, oc=oc, P=P)
    return pl.pallas_call(
        body,
        grid=(NPAD // RB,),
        in_specs=[
            pl.BlockSpec((RB, ic), lambda i: (i, 0)),
            pl.BlockSpec((ic, 2 * oc), lambda i: (0, 0)),
            pl.BlockSpec((2 * oc, 4), lambda i: (0, 0)),
        ],
        out_specs=[
            pl.BlockSpec((2, RB, P), lambda i: (0, i, 0)),
            pl.BlockSpec((4, RB), lambda i: (0, i)),
        ],
        out_shape=[
            jax.ShapeDtypeStruct((2, NPAD, P), jnp.float32),
            jax.ShapeDtypeStruct((4, NPAD), jnp.float32),
        ],
    )(x, w, amat)


def _node_matmul_next(prev_out, bias, w, amat, ocp, Pp, oc, P):
    body = functools.partial(_mm_next_body, ocp=ocp, oc=oc, P=P)
    return pl.pallas_call(
        body,
        grid=(NPAD // RB,),
        in_specs=[
            pl.BlockSpec((2, RB, Pp), lambda i: (0, i, 0)),
            pl.BlockSpec((1, 2 * ocp), lambda i: (0, 0)),
            pl.BlockSpec((2 * ocp, 2 * oc), lambda i: (0, 0)),
            pl.BlockSpec((2 * oc, 4), lambda i: (0, 0)),
        ],
        out_specs=[
            pl.BlockSpec((2, RB, P), lambda i: (0, i, 0)),
            pl.BlockSpec((4, RB), lambda i: (0, i)),
        ],
        out_shape=[
            jax.ShapeDtypeStruct((2, NPAD, P), jnp.float32),
            jax.ShapeDtypeStruct((4, NPAD), jnp.float32),
        ],
    )(prev_out, bias, w, amat)


def _final_node_body(o_ref, bias_ref, x_ref, *, ocp):
    x_ref[...] = _unpack_prev(o_ref[...], ocp, bias_ref[...])


def _final_node(prev_out, bias, ocp, Pp):
    body = functools.partial(_final_node_body, ocp=ocp)
    return pl.pallas_call(
        body,
        grid=(NPAD // RB,),
        in_specs=[
            pl.BlockSpec((2, RB, Pp), lambda i: (0, i, 0)),
            pl.BlockSpec((1, 2 * ocp), lambda i: (0, 0)),
        ],
        out_specs=pl.BlockSpec((RB, 2 * ocp), lambda i: (i, 0)),
        out_shape=jax.ShapeDtypeStruct((NPAD, 2 * ocp), jnp.float32),
    )(prev_out, bias)


def _mlp_body(x_ref, w1_ref, b1_ref, w2_ref, b2_ref, w3_ref, b3_ref,
              w4_ref, b4_ref, o_ref):
    h = x_ref[...]
    h = jnp.maximum(jnp.dot(h, w1_ref[...], preferred_element_type=jnp.float32)
                    + b1_ref[...], 0.0)
    h = jnp.maximum(jnp.dot(h, w2_ref[...], preferred_element_type=jnp.float32)
                    + b2_ref[...], 0.0)
    h = jnp.maximum(jnp.dot(h, w3_ref[...], preferred_element_type=jnp.float32)
                    + b3_ref[...], 0.0)
    o_ref[...] = (jnp.dot(h, w4_ref[...], preferred_element_type=jnp.float32)
                  + b4_ref[...])


def _mlp(x, lw1, lb1, lw2, lb2, lw3, lb3, lw4, lb4):
    m = x.shape[0]
    return pl.pallas_call(
        _mlp_body,
        in_specs=[pl.BlockSpec(x.shape, lambda: (0, 0))] + [
            spec for w, b in ((lw1, lb1), (lw2, lb2), (lw3, lb3), (lw4, lb4))
            for spec in (pl.BlockSpec(w.shape, lambda: (0, 0)),
                         pl.BlockSpec((1, b.shape[0]), lambda: (0, 0)))
        ],
        out_specs=pl.BlockSpec((m, lw4.shape[1]), lambda: (0, 0)),
        out_shape=jax.ShapeDtypeStruct((m, lw4.shape[1]), jnp.float32),
    )(x, lw1, lb1.reshape(1, -1), lw2, lb2.reshape(1, -1),
      lw3, lb3.reshape(1, -1), lw4, lb4.reshape(1, -1))


def _amat(att_src, att_dst):
    oc = att_src.shape[1]
    m = jnp.zeros((2 * oc, 4), jnp.float32)
    m = m.at[:oc, 0].set(att_src[0]).at[oc:, 1].set(att_src[1])
    m = m.at[:oc, 2].set(att_dst[0]).at[oc:, 3].set(att_dst[1])
    return m


def _sc_edge(h2flat, at4, srcp, dstp, P):
    mesh = plsc.VectorSubcoreMesh(core_axis_name="c", subcore_axis_name="s",
                                  num_cores=2, num_subcores=16)
    rstripe = NPAD // 16

    @functools.partial(
        pl.kernel,
        out_type=jax.ShapeDtypeStruct((2 * NPAD, P), jnp.float32),
        mesh=mesh,
        compiler_params=pltpu.CompilerParams(needs_layout_passes=False),
        scratch_types=[
            pltpu.VMEM_SHARED((NPAD,), jnp.float32),
            pltpu.VMEM_SHARED((NPAD,), jnp.float32),
            pltpu.VMEM_SHARED((NPAD, P), jnp.float32),
            pltpu.VMEM((CH,), jnp.float32),
            pltpu.VMEM((CH,), jnp.float32),
            pltpu.VMEM((CH,), jnp.int32),
            pltpu.VMEM((CH,), jnp.int32),
            pltpu.VMEM((CH,), jnp.int32),
            pltpu.VMEM((CH,), jnp.int32),
            pltpu.VMEM((CH,), jnp.int32),
            pltpu.VMEM((CH,), jnp.int32),
            pltpu.VMEM((CH,), jnp.int32),
            pltpu.VMEM((CH,), jnp.int32),
            pltpu.VMEM((CH,), jnp.float32),
            pltpu.VMEM((CH,), jnp.float32),
            pltpu.VMEM((CH, P), jnp.float32),
            pltpu.VMEM((CH, P), jnp.float32),
            pltpu.SemaphoreType.DMA(()),
            pltpu.SemaphoreType.DMA(()),
            pltpu.SemaphoreType.DMA(()),
            pltpu.SemaphoreType.DMA(()),
        ],
    )
    def k(h2_hbm, a_hbm, src_hbm, dst_hbm, out_hbm,
          sha_s, sha_d, acc, asg, adg, srcb0, srcb1, dstb0, dstb1,
          sidx0, sidx1, didx0, didx1, exbuf0, exbuf1, rows0, rows1,
          sem0, sem1, esem0, esem1):
        srcb = (srcb0, srcb1)
        dstb = (dstb0, dstb1)
        sidx = (sidx0, sidx1)
        didx = (didx0, didx1)
        exbuf = (exbuf0, exbuf1)
        rows = (rows0, rows1)
        sem = (sem0, sem1)
        esem = (esem0, esem1)
        c = lax.axis_index("c")
        s = lax.axis_index("s")
        ebase = s * EPT
        coff = c * NPAD

        @pl.when(s == 0)
        def _stage():
            pltpu.sync_copy(a_hbm.at[c], sha_s)
            pltpu.sync_copy(a_hbm.at[2 + c], sha_d)

        @pl.loop(0, CH)
        def _zb(i):
            for v in range(P // 16):
                rows[0][i, pl.ds(v * 16, 16)] = jnp.zeros((16,), jnp.float32)

        for kk in range(rstripe // CH):
            pltpu.sync_copy(rows[0], acc.at[pl.ds(s * rstripe + kk * CH, CH)])
        plsc.subcore_barrier()

        def edge_stage(j, b):
            pltpu.async_copy(src_hbm.at[pl.ds(ebase + j * CH, CH)],
                             srcb[b], esem[b])
            pltpu.async_copy(dst_hbm.at[pl.ds(ebase + j * CH, CH)],
                             dstb[b], esem[b])

        def prepare(j, b):
            pltpu.make_async_copy(src_hbm.at[pl.ds(0, CH)], srcb[b],
                                  esem[b]).wait()
            pltpu.make_async_copy(src_hbm.at[pl.ds(0, CH)], dstb[b],
                                  esem[b]).wait()

            @pl.loop(0, CH // 16, unroll=True)
            def _idx(g):
                sidx[b][pl.ds(g * 16, 16)] = srcb[b][pl.ds(g * 16, 16)] + coff
                didx[b][pl.ds(g * 16, 16)] = dstb[b][pl.ds(g * 16, 16)]

            pltpu.sync_copy(sha_s.at[srcb[b]], asg)
            pltpu.sync_copy(sha_d.at[dstb[b]], adg)

            @pl.loop(0, CH // 16, unroll=True)
            def _ex(g):
                al = asg[pl.ds(g * 16, 16)] + adg[pl.ds(g * 16, 16)]
                al = jnp.where(al >= 0, al, 0.2 * al)
                exbuf[b][pl.ds(g * 16, 16)] = jnp.exp(al)

            pltpu.async_copy(h2_hbm.at[sidx[b]], rows[b], sem[b])
            edge_stage(j + 2, b)

        def process(b):
            pltpu.make_async_copy(h2_hbm.at[pl.ds(0, CH)], rows[b],
                                  sem[b]).wait()

            @pl.loop(0, CH, unroll=4)
            def _scale(e):
                exb = plsc.load_gather(exbuf[b],
                                       [jnp.zeros((16,), jnp.int32) + e])
                for v in range(P // 16):
                    rows[b][e, pl.ds(v * 16, 16)] = (
                        rows[b][e, pl.ds(v * 16, 16)] * exb)

            pltpu.sync_copy(rows[b], acc.at[didx[b]], add=True)

        edge_stage(0, 0)
        edge_stage(1, 1)

        @pl.loop(0, NCH // 2)
        def _pair(t):
            j0 = 2 * t
            prepare(j0, 0)
            prepare(j0 + 1, 1)
            process(0)
            process(1)

        for b in (0, 1):
            pltpu.make_async_copy(src_hbm.at[pl.ds(0, CH)], srcb[b],
                                  esem[b]).wait()
            pltpu.make_async_copy(src_hbm.at[pl.ds(0, CH)], dstb[b],
                                  esem[b]).wait()

        plsc.subcore_barrier()
        pltpu.sync_copy(acc.at[pl.ds(s * rstripe, rstripe)],
                        out_hbm.at[pl.ds(coff + s * rstripe, rstripe)])

    return k(h2flat, at4, srcp, dstp)


def kernel(x, edge_index, batch, W1, as1, ad1, b1, W2, as2, ad2, b2,
           W3, as3, ad3, b3, W4, as4, ad4, b4,
           lw1, lb1, lw2, lb2, lw3, lb3, lw4, lb4):
    loop = jnp.arange(N, dtype=jnp.int32)
    dummy = jnp.full((EP + 2 * CH - E - N,), NPAD - 1, jnp.int32)
    srcp = jnp.concatenate([edge_index[0].astype(jnp.int32), loop, dummy])
    dstp = jnp.concatenate([edge_index[1].astype(jnp.int32), loop, dummy])

    xp = jnp.zeros((NPAD, x.shape[1]), x.dtype).at[:N].set(x)

    params = [(W1, as1, ad1, b1), (W2, as2, ad2, b2),
              (W3, as3, ad3, b3), (W4, as4, ad4, b4)]

    h2 = a = None
    prev = None
    for li, ((ic, oc, P), (W, asl, adl, bl)) in enumerate(zip(LAYERS, params)):
        am = _amat(asl, adl)
        if li == 0:
            h2, a = _node_matmul_first(xp, W, am, oc, P)
        else:
            ocp, Pp = LAYERS[li - 1][1], LAYERS[li - 1][2]
            h2, a = _node_matmul_next(prev, params[li - 1][3].reshape(1, -1),
                                      W, am, ocp, Pp, oc, P)
        out = _sc_edge(h2.reshape(2 * NPAD, P), a, srcp, dstp, P)
        prev = out.reshape(2, NPAD, P)

    oc4, P4 = LAYERS[3][1], LAYERS[3][2]
    x4 = _final_node(prev, b4.reshape(1, -1), oc4, P4)

    xr = x4[:N].reshape(1250, 480)
    xr = jnp.zeros((1280, 480), jnp.float32).at[:1250].set(xr)
    out = _mlp(xr, lw1, lb1, lw2, lb2, lw3, lb3, lw4, lb4)
    return out[:1250]


# speedup vs baseline: 44.4335x; 1.2201x over previous
import functools

import jax
import jax.numpy as jnp
from jax import lax
from jax.experimental import pallas as pl
from jax.experimental.pallas import tpu as pltpu
from jax.experimental.pallas import tpu_sc as plsc

N = 10000
E = 160000
NPAD = 10240
RB = 512
CH = 128
NCH = 84
EPT = NCH * CH
EP = 16 * EPT

LAYERS = [(336, 125, 128), (250, 75, 128), (150, 50, 128), (100, 30, 128)]


def _pack_h2(h, oc, P):
    rb = h.shape[0]
    ones = jnp.ones((rb, 1), jnp.float32)
    pad = jnp.zeros((rb, P - oc - 1), jnp.float32)
    h0 = jnp.concatenate([h[:, :oc], ones, pad], axis=1)
    h1 = jnp.concatenate([h[:, oc:], ones, pad], axis=1)
    return h0, h1


def _mm_first_body(x_ref, w_ref, amat_ref, h2_ref, a_ref, *, oc, P):
    h = jnp.dot(x_ref[...], w_ref[...], preferred_element_type=jnp.float32)
    a = jnp.dot(h, amat_ref[...], preferred_element_type=jnp.float32)
    a_ref[...] = a.T
    h0, h1 = _pack_h2(h, oc, P)
    h2_ref[0] = h0
    h2_ref[1] = h1


def _unpack_prev(o, ocp, bias):
    rb = o.shape[1]
    n0 = o[0, :, :ocp]
    d0 = jnp.broadcast_to(o[0, :, ocp:ocp + 1], (rb, ocp))
    n1 = o[1, :, :ocp]
    d1 = jnp.broadcast_to(o[1, :, ocp:ocp + 1], (rb, ocp))
    x = jnp.concatenate([n0 / (d0 + 1e-16), n1 / (d1 + 1e-16)], axis=1)
    return jnp.maximum(x + bias, 0.0)


def _mm_next_body(o_ref, bias_ref, w_ref, amat_ref, h2_ref, a_ref, *, ocp, oc, P):
    x = _unpack_prev(o_ref[...], ocp, bias_ref[...])
    h = jnp.dot(x, w_ref[...], preferred_element_type=jnp.float32)
    a = jnp.dot(h, amat_ref[...], preferred_element_type=jnp.float32)
    a_ref[...] = a.T
    h0, h1 = _pack_h2(h, oc, P)
    h2_ref[0] = h0
    h2_ref[1] = h1


def _node_matmul_first(x, w, amat, oc, P):
    ic = x.shape[1]
    body = functools.partial(_mm_first_body, oc=oc, P=P)
    return pl.pallas_call(
        body,
        grid=(NPAD // RB,),
        in_specs=[
            pl.BlockSpec((RB, ic), lambda i: (i, 0)),
            pl.BlockSpec((ic, 2 * oc), lambda i: (0, 0)),
            pl.BlockSpec((2 * oc, 4), lambda i: (0, 0)),
        ],
        out_specs=[
            pl.BlockSpec((2, RB, P), lambda i: (0, i, 0)),
            pl.BlockSpec((4, RB), lambda i: (0, i)),
        ],
        out_shape=[
            jax.ShapeDtypeStruct((2, NPAD, P), jnp.float32),
            jax.ShapeDtypeStruct((4, NPAD), jnp.float32),
        ],
    )(x, w, amat)


def _node_matmul_next(prev_out, bias, w, amat, ocp, Pp, oc, P):
    body = functools.partial(_mm_next_body, ocp=ocp, oc=oc, P=P)
    return pl.pallas_call(
        body,
        grid=(NPAD // RB,),
        in_specs=[
            pl.BlockSpec((2, RB, Pp), lambda i: (0, i, 0)),
            pl.BlockSpec((1, 2 * ocp), lambda i: (0, 0)),
            pl.BlockSpec((2 * ocp, 2 * oc), lambda i: (0, 0)),
            pl.BlockSpec((2 * oc, 4), lambda i: (0, 0)),
        ],
        out_specs=[
            pl.BlockSpec((2, RB, P), lambda i: (0, i, 0)),
            pl.BlockSpec((4, RB), lambda i: (0, i)),
        ],
        out_shape=[
            jax.ShapeDtypeStruct((2, NPAD, P), jnp.float32),
            jax.ShapeDtypeStruct((4, NPAD), jnp.float32),
        ],
    )(prev_out, bias, w, amat)


def _final_node_body(o_ref, bias_ref, x_ref, *, ocp):
    x_ref[...] = _unpack_prev(o_ref[...], ocp, bias_ref[...])


def _final_node(prev_out, bias, ocp, Pp):
    body = functools.partial(_final_node_body, ocp=ocp)
    return pl.pallas_call(
        body,
        grid=(NPAD // RB,),
        in_specs=[
            pl.BlockSpec((2, RB, Pp), lambda i: (0, i, 0)),
            pl.BlockSpec((1, 2 * ocp), lambda i: (0, 0)),
        ],
        out_specs=pl.BlockSpec((RB, 2 * ocp), lambda i: (i, 0)),
        out_shape=jax.ShapeDtypeStruct((NPAD, 2 * ocp), jnp.float32),
    )(prev_out, bias)


def _mlp_body(x_ref, w1_ref, b1_ref, w2_ref, b2_ref, w3_ref, b3_ref,
              w4_ref, b4_ref, o_ref):
    h = x_ref[...]
    h = jnp.maximum(jnp.dot(h, w1_ref[...], preferred_element_type=jnp.float32)
                    + b1_ref[...], 0.0)
    h = jnp.maximum(jnp.dot(h, w2_ref[...], preferred_element_type=jnp.float32)
                    + b2_ref[...], 0.0)
    h = jnp.maximum(jnp.dot(h, w3_ref[...], preferred_element_type=jnp.float32)
                    + b3_ref[...], 0.0)
    o_ref[...] = (jnp.dot(h, w4_ref[...], preferred_element_type=jnp.float32)
                  + b4_ref[...])


def _mlp(x, lw1, lb1, lw2, lb2, lw3, lb3, lw4, lb4):
    m = x.shape[0]
    return pl.pallas_call(
        _mlp_body,
        in_specs=[pl.BlockSpec(x.shape, lambda: (0, 0))] + [
            spec for w, b in ((lw1, lb1), (lw2, lb2), (lw3, lb3), (lw4, lb4))
            for spec in (pl.BlockSpec(w.shape, lambda: (0, 0)),
                         pl.BlockSpec((1, b.shape[0]), lambda: (0, 0)))
        ],
        out_specs=pl.BlockSpec((m, lw4.shape[1]), lambda: (0, 0)),
        out_shape=jax.ShapeDtypeStruct((m, lw4.shape[1]), jnp.float32),
    )(x, lw1, lb1.reshape(1, -1), lw2, lb2.reshape(1, -1),
      lw3, lb3.reshape(1, -1), lw4, lb4.reshape(1, -1))


def _amat(att_src, att_dst):
    oc = att_src.shape[1]
    m = jnp.zeros((2 * oc, 4), jnp.float32)
    m = m.at[:oc, 0].set(att_src[0]).at[oc:, 1].set(att_src[1])
    m = m.at[:oc, 2].set(att_dst[0]).at[oc:, 3].set(att_dst[1])
    return m


def _sc_edge(h2flat, at4, srcp, dstp, P):
    mesh = plsc.VectorSubcoreMesh(core_axis_name="c", subcore_axis_name="s",
                                  num_cores=2, num_subcores=16)
    rstripe = NPAD // 16

    @functools.partial(
        pl.kernel,
        out_type=jax.ShapeDtypeStruct((2 * NPAD, P), jnp.float32),
        mesh=mesh,
        compiler_params=pltpu.CompilerParams(needs_layout_passes=False),
        scratch_types=[
            pltpu.VMEM_SHARED((NPAD,), jnp.float32),
            pltpu.VMEM_SHARED((NPAD,), jnp.float32),
            pltpu.VMEM_SHARED((NPAD, P), jnp.float32),
            pltpu.VMEM((CH,), jnp.float32),
            pltpu.VMEM((CH,), jnp.float32),
            pltpu.VMEM((CH,), jnp.int32),
            pltpu.VMEM((CH,), jnp.int32),
            pltpu.VMEM((CH,), jnp.int32),
            pltpu.VMEM((CH,), jnp.int32),
            pltpu.VMEM((CH,), jnp.int32),
            pltpu.VMEM((CH,), jnp.int32),
            pltpu.VMEM((CH,), jnp.int32),
            pltpu.VMEM((CH,), jnp.int32),
            pltpu.VMEM((CH,), jnp.float32),
            pltpu.VMEM((CH,), jnp.float32),
            pltpu.VMEM((CH, P), jnp.float32),
            pltpu.VMEM((CH, P), jnp.float32),
            pltpu.SemaphoreType.DMA(()),
            pltpu.SemaphoreType.DMA(()),
            pltpu.SemaphoreType.DMA(()),
            pltpu.SemaphoreType.DMA(()),
            pltpu.SemaphoreType.DMA(()),
            pltpu.SemaphoreType.DMA(()),
            pltpu.SemaphoreType.DMA(()),
        ],
    )
    def k(h2_hbm, a_hbm, src_hbm, dst_hbm, out_hbm,
          sha_s, sha_d, acc, asg, adg, srcb0, srcb1, dstb0, dstb1,
          sidx0, sidx1, didx0, didx1, exbuf0, exbuf1, rows0, rows1,
          sem0, sem1, esem0, esem1, asem, ssem0, ssem1):
        srcb = (srcb0, srcb1)
        dstb = (dstb0, dstb1)
        sidx = (sidx0, sidx1)
        didx = (didx0, didx1)
        exbuf = (exbuf0, exbuf1)
        rows = (rows0, rows1)
        sem = (sem0, sem1)
        esem = (esem0, esem1)
        ssem = (ssem0, ssem1)
        c = lax.axis_index("c")
        s = lax.axis_index("s")
        ebase = s * EPT
        coff = c * NPAD

        @pl.when(s == 0)
        def _stage():
            pltpu.sync_copy(a_hbm.at[c], sha_s)
            pltpu.sync_copy(a_hbm.at[2 + c], sha_d)

        @pl.loop(0, CH)
        def _zb(i):
            for v in range(P // 16):
                rows[0][i, pl.ds(v * 16, 16)] = jnp.zeros((16,), jnp.float32)

        for kk in range(rstripe // CH):
            pltpu.sync_copy(rows[0], acc.at[pl.ds(s * rstripe + kk * CH, CH)])
        plsc.subcore_barrier()

        def edge_stage(j, b):
            pltpu.async_copy(src_hbm.at[pl.ds(ebase + j * CH, CH)],
                             srcb[b], esem[b])
            pltpu.async_copy(dst_hbm.at[pl.ds(ebase + j * CH, CH)],
                             dstb[b], esem[b])

        def prepare(j, b):
            pltpu.make_async_copy(src_hbm.at[pl.ds(0, CH)], srcb[b],
                                  esem[b]).wait()
            pltpu.make_async_copy(src_hbm.at[pl.ds(0, CH)], dstb[b],
                                  esem[b]).wait()

            @pl.loop(0, CH // 16, unroll=True)
            def _idx(g):
                sidx[b][pl.ds(g * 16, 16)] = srcb[b][pl.ds(g * 16, 16)] + coff
                didx[b][pl.ds(g * 16, 16)] = dstb[b][pl.ds(g * 16, 16)]

            pltpu.async_copy(h2_hbm.at[sidx[b]], rows[b], sem[b])
            pltpu.async_copy(sha_s.at[srcb[b]], asg, asem)
            pltpu.async_copy(sha_d.at[dstb[b]], adg, asem)
            pltpu.make_async_copy(src_hbm.at[pl.ds(0, CH)], asg, asem).wait()
            pltpu.make_async_copy(src_hbm.at[pl.ds(0, CH)], adg, asem).wait()

            @pl.loop(0, CH // 16, unroll=True)
            def _ex(g):
                al = asg[pl.ds(g * 16, 16)] + adg[pl.ds(g * 16, 16)]
                al = jnp.where(al >= 0, al, 0.2 * al)
                exbuf[b][pl.ds(g * 16, 16)] = jnp.exp(al)

            edge_stage(j + 2, b)

        def wait_scatter(b):
            pltpu.make_async_copy(h2_hbm.at[pl.ds(0, CH)], rows[b],
                                  ssem[b]).wait()

        def process(b):
            pltpu.make_async_copy(h2_hbm.at[pl.ds(0, CH)], rows[b],
                                  sem[b]).wait()

            @pl.loop(0, CH, unroll=4)
            def _scale(e):
                exb = plsc.load_gather(exbuf[b],
                                       [jnp.zeros((16,), jnp.int32) + e])
                for v in range(P // 16):
                    rows[b][e, pl.ds(v * 16, 16)] = (
                        rows[b][e, pl.ds(v * 16, 16)] * exb)

            pltpu.async_copy(rows[b], acc.at[didx[b]], ssem[b], add=True)

        edge_stage(0, 0)
        edge_stage(1, 1)
        prepare(0, 0)
        prepare(1, 1)
        process(0)
        process(1)

        @pl.loop(1, NCH // 2)
        def _pair(t):
            j0 = 2 * t
            wait_scatter(0)
            prepare(j0, 0)
            wait_scatter(1)
            prepare(j0 + 1, 1)
            process(0)
            process(1)

        wait_scatter(0)
        wait_scatter(1)
        for b in (0, 1):
            pltpu.make_async_copy(src_hbm.at[pl.ds(0, CH)], srcb[b],
                                  esem[b]).wait()
            pltpu.make_async_copy(src_hbm.at[pl.ds(0, CH)], dstb[b],
                                  esem[b]).wait()

        plsc.subcore_barrier()
        pltpu.sync_copy(acc.at[pl.ds(s * rstripe, rstripe)],
                        out_hbm.at[pl.ds(coff + s * rstripe, rstripe)])

    return k(h2flat, at4, srcp, dstp)


def kernel(x, edge_index, batch, W1, as1, ad1, b1, W2, as2, ad2, b2,
           W3, as3, ad3, b3, W4, as4, ad4, b4,
           lw1, lb1, lw2, lb2, lw3, lb3, lw4, lb4):
    loop = jnp.arange(N, dtype=jnp.int32)
    dummy = jnp.full((EP + 2 * CH - E - N,), NPAD - 1, jnp.int32)
    srcp = jnp.concatenate([edge_index[0].astype(jnp.int32), loop, dummy])
    dstp = jnp.concatenate([edge_index[1].astype(jnp.int32), loop, dummy])

    xp = jnp.zeros((NPAD, x.shape[1]), x.dtype).at[:N].set(x)

    params = [(W1, as1, ad1, b1), (W2, as2, ad2, b2),
              (W3, as3, ad3, b3), (W4, as4, ad4, b4)]

    h2 = a = None
    prev = None
    for li, ((ic, oc, P), (W, asl, adl, bl)) in enumerate(zip(LAYERS, params)):
        am = _amat(asl, adl)
        if li == 0:
            h2, a = _node_matmul_first(xp, W, am, oc, P)
        else:
            ocp, Pp = LAYERS[li - 1][1], LAYERS[li - 1][2]
            h2, a = _node_matmul_next(prev, params[li - 1][3].reshape(1, -1),
                                      W, am, ocp, Pp, oc, P)
        out = _sc_edge(h2.reshape(2 * NPAD, P), a, srcp, dstp, P)
        prev = out.reshape(2, NPAD, P)

    oc4, P4 = LAYERS[3][1], LAYERS[3][2]
    x4 = _final_node(prev, b4.reshape(1, -1), oc4, P4)

    xr = x4[:N].reshape(1250, 480)
    xr = jnp.zeros((1280, 480), jnp.float32).at[:1250].set(xr)
    out = _mlp(xr, lw1, lb1, lw2, lb2, lw3, lb3, lw4, lb4)
    return out[:1250]
